# baseline (speedup 1.0000x reference)
import functools

import jax
import jax.numpy as jnp
from jax import lax
from jax.experimental import pallas as pl
from jax.experimental.pallas import tpu as pltpu
from jax.experimental.pallas import tpu_sc as plsc

N = 10000
E = 320000
D_IN = 128
D_HID = 128
D_OUT = 64
K = 10

NC = 2
NS = 16
NW = NC * NS
CH = 192
CPT = 2 * (-(-E // (NW * CH * 2)))
E_PAD = NW * CH * CPT
N_PAD = 10240
RPT = N_PAD // NS
DUMMY = N


@functools.cache
def _make_sc_propagate():
    sc_mesh = plsc.VectorSubcoreMesh(
        core_axis_name="c", subcore_axis_name="s", num_cores=NC, num_subcores=NS
    )
    return pl.kernel(
        _sc_propagate_body,
        out_type=jax.ShapeDtypeStruct((NC * N_PAD, D_OUT), jnp.float32),
        mesh=sc_mesh,
        scratch_types=[
            pltpu.VMEM((CPT, CH), jnp.int32),
            pltpu.VMEM((CPT, CH), jnp.int32),
            pltpu.VMEM((CH, D_OUT), jnp.float32),
            pltpu.VMEM((CH, D_OUT), jnp.float32),
            pltpu.VMEM_SHARED((N_PAD, D_OUT), jnp.float32),
            pltpu.VMEM_SHARED((N_PAD, D_OUT), jnp.float32),
            pltpu.SemaphoreType.DMA,
            pltpu.SemaphoreType.DMA,
        ],
        compiler_params=pltpu.CompilerParams(use_tc_tiling_on_sc=False),
    )


def _sc_propagate(*args):
    return _make_sc_propagate()(*args)


def _sc_propagate_body(src_hbm, zeros_hbm, rows_hbm, cols_hbm, parts_hbm,
                       row_idx, col_idx, rows_a, rows_b, acc, table,
                       sem_a, sem_b):
    c = lax.axis_index("c")
    s = lax.axis_index("s")
    w = s * NC + c

    pltpu.sync_copy(rows_hbm.at[pl.ds(w * CPT, CPT)], row_idx)
    pltpu.sync_copy(cols_hbm.at[pl.ds(w * CPT, CPT)], col_idx)
    pltpu.sync_copy(src_hbm.at[pl.ds(s * RPT, RPT)],
                    table.at[pl.ds(s * RPT, RPT)])

    @pl.when(c == 0)
    def _():
        pltpu.sync_copy(src_hbm.at[pl.ds(s * RPT, RPT)],
                        acc.at[pl.ds(s * RPT, RPT)])

    @pl.when(c != 0)
    def _():
        pltpu.sync_copy(zeros_hbm.at[pl.ds(s * RPT, RPT)],
                        acc.at[pl.ds(s * RPT, RPT)])

    plsc.subcore_barrier()

    pltpu.async_copy(table.at[row_idx.at[0]], rows_a, sem_a)

    def chunk_pair(j, _):
        i0 = 2 * j
        pltpu.async_copy(table.at[row_idx.at[i0 + 1]], rows_b, sem_b)
        pltpu.make_async_copy(table.at[row_idx.at[i0]], rows_a, sem_a).wait()
        pltpu.sync_copy(rows_a, acc.at[col_idx.at[i0]], add=True)

        @pl.when(i0 + 2 < CPT)
        def _():
            pltpu.async_copy(table.at[row_idx.at[i0 + 2]], rows_a, sem_a)

        pltpu.make_async_copy(table.at[row_idx.at[i0 + 1]], rows_b,
                              sem_b).wait()
        pltpu.sync_copy(rows_b, acc.at[col_idx.at[i0 + 1]], add=True)
        return 0

    lax.fori_loop(0, CPT // 2, chunk_pair, 0)

    plsc.subcore_barrier()

    pltpu.sync_copy(acc.at[pl.ds(s * RPT, RPT)],
                    parts_hbm.at[pl.ds(c * N_PAD + s * RPT, RPT)])


DW = 8


@functools.cache
def _make_sc_degree():
    sc_mesh = plsc.VectorSubcoreMesh(
        core_axis_name="c", subcore_axis_name="s", num_cores=NC, num_subcores=NS
    )
    return pl.kernel(
        _sc_degree_body,
        out_type=jax.ShapeDtypeStruct((NC * N_PAD, DW), jnp.float32),
        mesh=sc_mesh,
        scratch_types=[
            pltpu.VMEM((CPT, CH), jnp.int32),
            pltpu.VMEM((CH, DW), jnp.float32),
            pltpu.VMEM_SHARED((N_PAD, DW), jnp.float32),
        ],
        compiler_params=pltpu.CompilerParams(use_tc_tiling_on_sc=False),
    )


def _sc_degree(*args):
    return _make_sc_degree()(*args)


def _sc_degree_body(ones_hbm, zeros_hbm, cols_hbm, parts_hbm,
                    col_idx, ones_v, acc):
    c = lax.axis_index("c")
    s = lax.axis_index("s")
    w = s * NC + c

    pltpu.sync_copy(cols_hbm.at[pl.ds(w * CPT, CPT)], col_idx)
    pltpu.sync_copy(ones_hbm.at[pl.ds(0, CH)], ones_v)

    @pl.when(c == 0)
    def _():
        pltpu.sync_copy(ones_hbm.at[pl.ds(s * RPT, RPT)],
                        acc.at[pl.ds(s * RPT, RPT)])

    @pl.when(c != 0)
    def _():
        pltpu.sync_copy(zeros_hbm.at[pl.ds(s * RPT, RPT)],
                        acc.at[pl.ds(s * RPT, RPT)])

    plsc.subcore_barrier()

    def chunk(i, _):
        pltpu.sync_copy(ones_v, acc.at[col_idx.at[i]], add=True)
        return 0

    lax.fori_loop(0, CPT, chunk, 0)

    plsc.subcore_barrier()

    pltpu.sync_copy(acc.at[pl.ds(s * RPT, RPT)],
                    parts_hbm.at[pl.ds(c * N_PAD + s * RPT, RPT)])


def _mlp_body(x_ref, w1_ref, b1_ref, w2_ref, b2_ref, h_ref):
    a = jnp.dot(x_ref[...], w1_ref[...], preferred_element_type=jnp.float32)
    a = jnp.maximum(a + b1_ref[...], 0.0)
    h_ref[...] = (
        jnp.dot(a, w2_ref[...], preferred_element_type=jnp.float32) + b2_ref[...]
    )


def _dinv_body(p0_ref, p1_ref, h_ref, dinv_ref, s0_ref):
    deg = p0_ref[...] + p1_ref[...]
    dinv = jnp.where(deg > 0.0, lax.rsqrt(deg), 0.0)[:, 0:1]
    dinv_ref[...] = dinv * jnp.ones((1, D_OUT), jnp.float32)
    s0_ref[...] = dinv * h_ref[...]


def _combine_body(p0_ref, p1_ref, dinv_ref, cur_ref, s_ref):
    t = dinv_ref[...] * (p0_ref[...] + p1_ref[...])
    cur_ref[...] = t
    s_ref[...] = dinv_ref[...] * t


def _retention_body(*refs):
    pred_refs = refs[: K + 1]
    wp_ref, bp_ref, out_ref = refs[K + 1], refs[K + 2], refs[K + 3]
    acc = jnp.zeros(out_ref.shape, out_ref.dtype)
    for p_ref in pred_refs:
        p = p_ref[...]
        sc = jnp.sum(p * wp_ref[...], axis=1, keepdims=True) + bp_ref[...]
        sg = 1.0 / (1.0 + jnp.exp(-sc))
        acc = acc + sg * p
    out_ref[...] = acc


_BN = 80
_NBP = N_PAD // _BN
_NBN = N // _BN


def _row_spec(d):
    return pl.BlockSpec((_BN, d), lambda i: (i, 0))


def _p1_spec():
    return pl.BlockSpec((_BN, D_OUT), lambda i: (i + _NBP, 0))


def _full_spec(r, c):
    return pl.BlockSpec((r, c), lambda i: (0, 0))


def kernel(x, edge_index, W1, b1, W2, b2, Wp, bp):
    f32 = jnp.float32
    row = edge_index[0]
    col = edge_index[1]
    colp = jnp.where(row == col, DUMMY, col).astype(jnp.int32)
    pad = E_PAD - E
    rows_full = jnp.concatenate([row.astype(jnp.int32),
                                 jnp.zeros((pad,), jnp.int32)]
                                ).reshape(NW * CPT, CH)
    cols_full = jnp.concatenate([colp, jnp.full((pad,), DUMMY, jnp.int32)]
                                ).reshape(NW * CPT, CH)
    zeros_pd = jnp.zeros((N_PAD, D_OUT), f32)
    zeros_dw = jnp.zeros((N_PAD, DW), f32)
    ones_dw = jnp.ones((N_PAD, DW), f32)

    h = pl.pallas_call(
        _mlp_body,
        grid=(_NBN,),
        in_specs=[
            _row_spec(D_IN),
            _full_spec(D_IN, D_HID),
            _full_spec(1, D_HID),
            _full_spec(D_HID, D_OUT),
            _full_spec(1, D_OUT),
        ],
        out_specs=_row_spec(D_OUT),
        out_shape=jax.ShapeDtypeStruct((N, D_OUT), f32),
    )(x, W1, b1.reshape(1, D_HID), W2, b2.reshape(1, D_OUT))
    h_pd = jnp.pad(h, ((0, N_PAD - N), (0, 0)))

    deg_parts = _sc_degree(ones_dw, zeros_dw, cols_full)

    dinv, cur_s = pl.pallas_call(
        _dinv_body,
        grid=(_NBP,),
        in_specs=[
            pl.BlockSpec((_BN, DW), lambda i: (i, 0)),
            pl.BlockSpec((_BN, DW), lambda i: (i + _NBP, 0)),
            _row_spec(D_OUT),
        ],
        out_specs=[_row_spec(D_OUT), _row_spec(D_OUT)],
        out_shape=[
            jax.ShapeDtypeStruct((N_PAD, D_OUT), f32),
            jax.ShapeDtypeStruct((N_PAD, D_OUT), f32),
        ],
    )(deg_parts, deg_parts, h_pd)

    preds = [h]
    for _ in range(K):
        parts = _sc_propagate(cur_s, zeros_pd, rows_full, cols_full)
        cur, cur_s = pl.pallas_call(
            _combine_body,
            grid=(_NBP,),
            in_specs=[_row_spec(D_OUT), _p1_spec(), _row_spec(D_OUT)],
            out_specs=[_row_spec(D_OUT), _row_spec(D_OUT)],
            out_shape=[
                jax.ShapeDtypeStruct((N_PAD, D_OUT), f32),
                jax.ShapeDtypeStruct((N_PAD, D_OUT), f32),
            ],
        )(parts, parts, dinv)
        preds.append(cur)

    out = pl.pallas_call(
        _retention_body,
        grid=(_NBN,),
        in_specs=[_row_spec(D_OUT)] * (K + 1)
        + [_full_spec(1, D_OUT), _full_spec(1, 1)],
        out_specs=_row_spec(D_OUT),
        out_shape=jax.ShapeDtypeStruct((N, D_OUT), f32),
    )(*preds, Wp.reshape(1, D_OUT), bp.reshape(1, 1))
    return out


# speedup vs baseline: 1.2997x; 1.2997x over previous
import functools

import jax
import jax.numpy as jnp
from jax import lax
from jax.experimental import pallas as pl
from jax.experimental.pallas import tpu as pltpu
from jax.experimental.pallas import tpu_sc as plsc

N = 10000
E = 320000
D_IN = 128
D_HID = 128
D_OUT = 64
K = 10

NC = 2
NS = 16
NW = NC * NS
CH = 192
CPT = 2 * (-(-E // (NW * CH * 2)))
E_PAD = NW * CH * CPT
N_PAD = 10240
RPT = N_PAD // NS
DUMMY = N


SUB = 80
MAXPOLL = 512


@functools.cache
def _make_sc_fused():
    sc_mesh = plsc.VectorSubcoreMesh(
        core_axis_name="c", subcore_axis_name="s", num_cores=NC, num_subcores=NS
    )
    return pl.kernel(
        _sc_fused_body,
        out_type=(
            jax.ShapeDtypeStruct((K * N_PAD, D_OUT), jnp.float32),
            jax.ShapeDtypeStruct((4 * N_PAD, D_OUT), jnp.float32),
            jax.ShapeDtypeStruct((32,), jnp.int32),
        ),
        mesh=sc_mesh,
        scratch_types=[
            pltpu.VMEM((CPT, CH), jnp.int32),
            pltpu.VMEM((CPT, CH), jnp.int32),
            pltpu.VMEM((CH, D_OUT), jnp.float32),
            pltpu.VMEM((CH, D_OUT), jnp.float32),
            pltpu.VMEM((16,), jnp.int32),
            pltpu.VMEM((16,), jnp.int32),
            pltpu.SMEM((1,), jnp.int32),
            pltpu.VMEM_SHARED((N_PAD, D_OUT), jnp.float32),
            pltpu.VMEM_SHARED((N_PAD, D_OUT), jnp.float32),
            pltpu.SemaphoreType.DMA,
            pltpu.SemaphoreType.DMA,
        ],
        compiler_params=pltpu.CompilerParams(use_tc_tiling_on_sc=False,
                                             needs_layout_passes=False),
    )


def _sc_fused(*args):
    return _make_sc_fused()(*args)


def _edge_phase(table, acc, row_idx, col_idx, rows_a, rows_b, sem_a, sem_b):
    pltpu.async_copy(table.at[row_idx.at[0]], rows_a.at[pl.ds(0, CH)], sem_a)

    def chunk_pair(j, _):
        i0 = 2 * j
        pltpu.async_copy(table.at[row_idx.at[i0 + 1]], rows_b.at[pl.ds(0, CH)],
                         sem_b)
        pltpu.make_async_copy(table.at[row_idx.at[i0]],
                              rows_a.at[pl.ds(0, CH)], sem_a).wait()
        pltpu.sync_copy(rows_a.at[pl.ds(0, CH)], acc.at[col_idx.at[i0]],
                        add=True)

        @pl.when(i0 + 2 < CPT)
        def _():
            pltpu.async_copy(table.at[row_idx.at[i0 + 2]],
                             rows_a.at[pl.ds(0, CH)], sem_a)

        pltpu.make_async_copy(table.at[row_idx.at[i0 + 1]],
                              rows_b.at[pl.ds(0, CH)], sem_b).wait()
        pltpu.sync_copy(rows_b.at[pl.ds(0, CH)], acc.at[col_idx.at[i0 + 1]],
                        add=True)
        return 0

    lax.fori_loop(0, CPT // 2, chunk_pair, 0)


def _sc_fused_body(src_hbm, zeros_hbm, dinv_hbm, rows_hbm, cols_hbm,
                   preds_hbm, parts_hbm, flags_hbm,
                   row_idx, col_idx, rows_a, rows_b, flagw, flagr, done,
                   acc, table, sem_a, sem_b):
    c = lax.axis_index("c")
    s = lax.axis_index("s")
    w = s * NC + c
    base = s * RPT

    @pl.when(s == 0)
    def _():
        flagw[...] = jnp.zeros((16,), jnp.int32)
        pltpu.sync_copy(flagw, flags_hbm.at[pl.ds(c * 16, 16)])

    pltpu.sync_copy(rows_hbm.at[pl.ds(w * CPT, CPT)], row_idx)
    pltpu.sync_copy(cols_hbm.at[pl.ds(w * CPT, CPT)], col_idx)
    pltpu.sync_copy(src_hbm.at[pl.ds(base, RPT)], table.at[pl.ds(base, RPT)])

    @pl.when(c == 0)
    def _():
        pltpu.sync_copy(src_hbm.at[pl.ds(base, RPT)], acc.at[pl.ds(base, RPT)])

    @pl.when(c != 0)
    def _():
        pltpu.sync_copy(zeros_hbm.at[pl.ds(base, RPT)],
                        acc.at[pl.ds(base, RPT)])

    plsc.subcore_barrier()

    for k in range(1, K + 1):
        _edge_phase(table, acc, row_idx, col_idx, rows_a, rows_b, sem_a, sem_b)
        plsc.subcore_barrier()

        pofs = ((k % 2) * 2 + c) * N_PAD
        pltpu.sync_copy(acc.at[pl.ds(base, RPT)],
                        parts_hbm.at[pl.ds(pofs + base, RPT)])
        plsc.subcore_barrier()

        @pl.when(s == 0)
        def _():
            flagw[...] = jnp.full((16,), k, jnp.int32)
            pltpu.sync_copy(flagw, flags_hbm.at[pl.ds(c * 16, 16)])

        other = 1 - c
        done[0] = 0

        def poll_body(t, _):
            @pl.when(done[0] == 0)
            def _():
                pltpu.sync_copy(flags_hbm.at[pl.ds(other * 16, 16)], flagr)
                m = jnp.min(flagr[...])
                done[0] = (m == k).astype(jnp.int32)
            return 0

        lax.fori_loop(0, MAXPOLL, poll_body, 0)

        p0ofs = (k % 2) * 2 * N_PAD
        p1ofs = p0ofs + N_PAD

        def combine_block(b, _):
            o = base + b * SUB
            pltpu.sync_copy(parts_hbm.at[pl.ds(p0ofs + o, SUB)],
                            rows_a.at[pl.ds(0, SUB)])
            pltpu.sync_copy(parts_hbm.at[pl.ds(p1ofs + o, SUB)],
                            rows_b.at[pl.ds(0, SUB)])
            pltpu.sync_copy(dinv_hbm.at[pl.ds(o, SUB)],
                            rows_a.at[pl.ds(96, SUB)])

            def rowfn(r, _):
                for g in range(4):
                    sl = pl.ds(g * 16, 16)
                    d = rows_a[r + 96, sl]
                    cur = d * (rows_a[r, sl] + rows_b[r, sl])
                    rows_b[r, sl] = cur
                    rows_a[r, sl] = d * cur
                return 0

            lax.fori_loop(0, SUB, rowfn, 0)

            pltpu.sync_copy(rows_b.at[pl.ds(0, SUB)],
                            preds_hbm.at[pl.ds((k - 1) * N_PAD + o, SUB)])
            pltpu.sync_copy(rows_a.at[pl.ds(0, SUB)], table.at[pl.ds(o, SUB)])
            if k < K:
                @pl.when(c == 0)
                def _():
                    pltpu.sync_copy(rows_a.at[pl.ds(0, SUB)],
                                    acc.at[pl.ds(o, SUB)])
            return 0

        lax.fori_loop(0, RPT // SUB, combine_block, 0)

        if k < K:
            @pl.when(c != 0)
            def _():
                pltpu.sync_copy(zeros_hbm.at[pl.ds(base, RPT)],
                                acc.at[pl.ds(base, RPT)])
        plsc.subcore_barrier()


DW = 8


@functools.cache
def _make_sc_degree():
    sc_mesh = plsc.VectorSubcoreMesh(
        core_axis_name="c", subcore_axis_name="s", num_cores=NC, num_subcores=NS
    )
    return pl.kernel(
        _sc_degree_body,
        out_type=jax.ShapeDtypeStruct((NC * N_PAD, DW), jnp.float32),
        mesh=sc_mesh,
        scratch_types=[
            pltpu.VMEM((CPT, CH), jnp.int32),
            pltpu.VMEM((CH, DW), jnp.float32),
            pltpu.VMEM_SHARED((N_PAD, DW), jnp.float32),
        ],
        compiler_params=pltpu.CompilerParams(use_tc_tiling_on_sc=False),
    )


def _sc_degree(*args):
    return _make_sc_degree()(*args)


def _sc_degree_body(ones_hbm, zeros_hbm, cols_hbm, parts_hbm,
                    col_idx, ones_v, acc):
    c = lax.axis_index("c")
    s = lax.axis_index("s")
    w = s * NC + c

    pltpu.sync_copy(cols_hbm.at[pl.ds(w * CPT, CPT)], col_idx)
    pltpu.sync_copy(ones_hbm.at[pl.ds(0, CH)], ones_v)

    @pl.when(c == 0)
    def _():
        pltpu.sync_copy(ones_hbm.at[pl.ds(s * RPT, RPT)],
                        acc.at[pl.ds(s * RPT, RPT)])

    @pl.when(c != 0)
    def _():
        pltpu.sync_copy(zeros_hbm.at[pl.ds(s * RPT, RPT)],
                        acc.at[pl.ds(s * RPT, RPT)])

    plsc.subcore_barrier()

    def chunk(i, _):
        pltpu.sync_copy(ones_v, acc.at[col_idx.at[i]], add=True)
        return 0

    lax.fori_loop(0, CPT, chunk, 0)

    plsc.subcore_barrier()

    pltpu.sync_copy(acc.at[pl.ds(s * RPT, RPT)],
                    parts_hbm.at[pl.ds(c * N_PAD + s * RPT, RPT)])


def _mlp_body(x_ref, w1_ref, b1_ref, w2_ref, b2_ref, h_ref):
    a = jnp.dot(x_ref[...], w1_ref[...], preferred_element_type=jnp.float32)
    a = jnp.maximum(a + b1_ref[...], 0.0)
    h_ref[...] = (
        jnp.dot(a, w2_ref[...], preferred_element_type=jnp.float32) + b2_ref[...]
    )


def _dinv_body(p0_ref, p1_ref, h_ref, dinv_ref, s0_ref):
    deg = p0_ref[...] + p1_ref[...]
    dinv = jnp.where(deg > 0.0, lax.rsqrt(deg), 0.0)[:, 0:1]
    dinv_ref[...] = dinv * jnp.ones((1, D_OUT), jnp.float32)
    s0_ref[...] = dinv * h_ref[...]


def _combine_body(p0_ref, p1_ref, dinv_ref, cur_ref, s_ref):
    t = dinv_ref[...] * (p0_ref[...] + p1_ref[...])
    cur_ref[...] = t
    s_ref[...] = dinv_ref[...] * t


def _retention_body(*refs):
    pred_refs = refs[: K + 1]
    wp_ref, bp_ref, out_ref = refs[K + 1], refs[K + 2], refs[K + 3]
    acc = jnp.zeros(out_ref.shape, out_ref.dtype)
    for p_ref in pred_refs:
        p = p_ref[...]
        sc = jnp.sum(p * wp_ref[...], axis=1, keepdims=True) + bp_ref[...]
        sg = 1.0 / (1.0 + jnp.exp(-sc))
        acc = acc + sg * p
    out_ref[...] = acc


_BN = 80
_NBP = N_PAD // _BN
_NBN = N // _BN


def _row_spec(d):
    return pl.BlockSpec((_BN, d), lambda i: (i, 0))


def _p1_spec():
    return pl.BlockSpec((_BN, D_OUT), lambda i: (i + _NBP, 0))


def _full_spec(r, c):
    return pl.BlockSpec((r, c), lambda i: (0, 0))


def kernel(x, edge_index, W1, b1, W2, b2, Wp, bp):
    f32 = jnp.float32
    row = edge_index[0]
    col = edge_index[1]
    colp = jnp.where(row == col, DUMMY, col).astype(jnp.int32)
    pad = E_PAD - E
    rows_full = jnp.concatenate([row.astype(jnp.int32),
                                 jnp.zeros((pad,), jnp.int32)]
                                ).reshape(NW * CPT, CH)
    cols_full = jnp.concatenate([colp, jnp.full((pad,), DUMMY, jnp.int32)]
                                ).reshape(NW * CPT, CH)
    zeros_pd = jnp.zeros((N_PAD, D_OUT), f32)
    zeros_dw = jnp.zeros((N_PAD, DW), f32)
    ones_dw = jnp.ones((N_PAD, DW), f32)

    h = pl.pallas_call(
        _mlp_body,
        grid=(_NBN,),
        in_specs=[
            _row_spec(D_IN),
            _full_spec(D_IN, D_HID),
            _full_spec(1, D_HID),
            _full_spec(D_HID, D_OUT),
            _full_spec(1, D_OUT),
        ],
        out_specs=_row_spec(D_OUT),
        out_shape=jax.ShapeDtypeStruct((N, D_OUT), f32),
    )(x, W1, b1.reshape(1, D_HID), W2, b2.reshape(1, D_OUT))
    h_pd = jnp.pad(h, ((0, N_PAD - N), (0, 0)))

    deg_parts = _sc_degree(ones_dw, zeros_dw, cols_full)

    dinv, cur_s = pl.pallas_call(
        _dinv_body,
        grid=(_NBP,),
        in_specs=[
            pl.BlockSpec((_BN, DW), lambda i: (i, 0)),
            pl.BlockSpec((_BN, DW), lambda i: (i + _NBP, 0)),
            _row_spec(D_OUT),
        ],
        out_specs=[_row_spec(D_OUT), _row_spec(D_OUT)],
        out_shape=[
            jax.ShapeDtypeStruct((N_PAD, D_OUT), f32),
            jax.ShapeDtypeStruct((N_PAD, D_OUT), f32),
        ],
    )(deg_parts, deg_parts, h_pd)

    preds_out, _parts, _flags = _sc_fused(cur_s, zeros_pd, dinv,
                                          rows_full, cols_full)
    preds = [h]

    sec = N_PAD // _BN

    def _pred_spec(k):
        return pl.BlockSpec((_BN, D_OUT), lambda i, kk=k: (i + kk * sec, 0))

    out = pl.pallas_call(
        _retention_body,
        grid=(_NBN,),
        in_specs=[_row_spec(D_OUT)]
        + [_pred_spec(k) for k in range(K)]
        + [_full_spec(1, D_OUT), _full_spec(1, 1)],
        out_specs=_row_spec(D_OUT),
        out_shape=jax.ShapeDtypeStruct((N, D_OUT), f32),
    )(h, *([preds_out] * K), Wp.reshape(1, D_OUT), bp.reshape(1, 1))
    return out


# speedup vs baseline: 1.3093x; 1.0073x over previous
import functools

import jax
import jax.numpy as jnp
from jax import lax
from jax.experimental import pallas as pl
from jax.experimental.pallas import tpu as pltpu
from jax.experimental.pallas import tpu_sc as plsc

N = 10000
E = 320000
D_IN = 128
D_HID = 128
D_OUT = 64
K = 10

NC = 2
NS = 16
NW = NC * NS
CH = 192
CPT = 2 * (-(-E // (NW * CH * 2)))
E_PAD = NW * CH * CPT
N_PAD = 10240
RPT = N_PAD // NS
DUMMY = N


SUB = 80
MAXPOLL = 512


@functools.cache
def _make_sc_fused():
    sc_mesh = plsc.VectorSubcoreMesh(
        core_axis_name="c", subcore_axis_name="s", num_cores=NC, num_subcores=NS
    )
    return pl.kernel(
        _sc_fused_body,
        out_type=(
            jax.ShapeDtypeStruct((K * N_PAD, D_OUT), jnp.float32),
            jax.ShapeDtypeStruct((4 * N_PAD, D_OUT), jnp.float32),
            jax.ShapeDtypeStruct((32,), jnp.int32),
        ),
        mesh=sc_mesh,
        scratch_types=[
            pltpu.VMEM((CPT, CH), jnp.int32),
            pltpu.VMEM((CPT, CH), jnp.int32),
            pltpu.VMEM((CH, D_OUT), jnp.float32),
            pltpu.VMEM((CH, D_OUT), jnp.float32),
            pltpu.VMEM((16,), jnp.int32),
            pltpu.VMEM((16,), jnp.int32),
            pltpu.SMEM((1,), jnp.int32),
            pltpu.VMEM_SHARED((N_PAD, D_OUT), jnp.float32),
            pltpu.VMEM_SHARED((N_PAD, D_OUT), jnp.float32),
            pltpu.SemaphoreType.DMA,
            pltpu.SemaphoreType.DMA,
        ],
        compiler_params=pltpu.CompilerParams(use_tc_tiling_on_sc=False,
                                             needs_layout_passes=False),
    )


def _sc_fused(*args):
    return _make_sc_fused()(*args)


def _edge_phase(table, acc, row_idx, col_idx, rows_a, rows_b, sem_a, sem_b):
    pltpu.async_copy(table.at[row_idx.at[0]], rows_a.at[pl.ds(0, CH)], sem_a)

    def chunk_pair(j, _):
        i0 = 2 * j
        pltpu.async_copy(table.at[row_idx.at[i0 + 1]], rows_b.at[pl.ds(0, CH)],
                         sem_b)
        pltpu.make_async_copy(table.at[row_idx.at[i0]],
                              rows_a.at[pl.ds(0, CH)], sem_a).wait()
        pltpu.sync_copy(rows_a.at[pl.ds(0, CH)], acc.at[col_idx.at[i0]],
                        add=True)

        @pl.when(i0 + 2 < CPT)
        def _():
            pltpu.async_copy(table.at[row_idx.at[i0 + 2]],
                             rows_a.at[pl.ds(0, CH)], sem_a)

        pltpu.make_async_copy(table.at[row_idx.at[i0 + 1]],
                              rows_b.at[pl.ds(0, CH)], sem_b).wait()
        pltpu.sync_copy(rows_b.at[pl.ds(0, CH)], acc.at[col_idx.at[i0 + 1]],
                        add=True)
        return 0

    lax.fori_loop(0, CPT // 2, chunk_pair, 0)


def _sc_fused_body(src_hbm, zeros_hbm, dinv_hbm, rows_hbm, cols_hbm,
                   preds_hbm, parts_hbm, flags_hbm,
                   row_idx, col_idx, rows_a, rows_b, flagw, flagr, done,
                   acc, table, sem_a, sem_b):
    c = lax.axis_index("c")
    s = lax.axis_index("s")
    w = s * NC + c
    base = s * RPT

    @pl.when(s == 0)
    def _():
        flagw[...] = jnp.zeros((16,), jnp.int32)
        pltpu.sync_copy(flagw, flags_hbm.at[pl.ds(c * 16, 16)])

    pltpu.sync_copy(rows_hbm.at[pl.ds(w * CPT, CPT)], row_idx)
    pltpu.sync_copy(cols_hbm.at[pl.ds(w * CPT, CPT)], col_idx)
    pltpu.sync_copy(src_hbm.at[pl.ds(base, RPT)], table.at[pl.ds(base, RPT)])

    @pl.when(c == 0)
    def _():
        pltpu.sync_copy(src_hbm.at[pl.ds(base, RPT)], acc.at[pl.ds(base, RPT)])

    @pl.when(c != 0)
    def _():
        pltpu.sync_copy(zeros_hbm.at[pl.ds(base, RPT)],
                        acc.at[pl.ds(base, RPT)])

    plsc.subcore_barrier()

    for k in range(1, K + 1):
        _edge_phase(table, acc, row_idx, col_idx, rows_a, rows_b, sem_a, sem_b)
        plsc.subcore_barrier()

        pofs = ((k % 2) * 2 + c) * N_PAD
        pltpu.sync_copy(acc.at[pl.ds(base, RPT)],
                        parts_hbm.at[pl.ds(pofs + base, RPT)])
        plsc.subcore_barrier()

        @pl.when(s == 0)
        def _():
            flagw[...] = jnp.full((16,), k, jnp.int32)
            pltpu.sync_copy(flagw, flags_hbm.at[pl.ds(c * 16, 16)])

        other = 1 - c
        done[0] = 0

        def poll_body(t, _):
            @pl.when(done[0] == 0)
            def _():
                pltpu.sync_copy(flags_hbm.at[pl.ds(other * 16, 16)], flagr)
                m = jnp.min(flagr[...])
                done[0] = (m == k).astype(jnp.int32)
            return 0

        lax.fori_loop(0, MAXPOLL, poll_body, 0)

        p0ofs = (k % 2) * 2 * N_PAD
        p1ofs = p0ofs + N_PAD

        def combine_block(b, _):
            o = base + b * SUB
            pltpu.sync_copy(parts_hbm.at[pl.ds(p0ofs + o, SUB)],
                            rows_a.at[pl.ds(0, SUB)])
            pltpu.sync_copy(parts_hbm.at[pl.ds(p1ofs + o, SUB)],
                            rows_b.at[pl.ds(0, SUB)])
            pltpu.sync_copy(dinv_hbm.at[pl.ds(o, SUB)],
                            rows_a.at[pl.ds(96, SUB)])

            def rowfn(r, _):
                for g in range(4):
                    sl = pl.ds(g * 16, 16)
                    d = rows_a[r + 96, sl]
                    cur = d * (rows_a[r, sl] + rows_b[r, sl])
                    rows_b[r, sl] = cur
                    rows_a[r, sl] = d * cur
                return 0

            lax.fori_loop(0, SUB, rowfn, 0)

            @pl.when(c == 0)
            def _():
                pltpu.sync_copy(rows_b.at[pl.ds(0, SUB)],
                                preds_hbm.at[pl.ds((k - 1) * N_PAD + o, SUB)])
            pltpu.sync_copy(rows_a.at[pl.ds(0, SUB)], table.at[pl.ds(o, SUB)])
            if k < K:
                @pl.when(c == 0)
                def _():
                    pltpu.sync_copy(rows_a.at[pl.ds(0, SUB)],
                                    acc.at[pl.ds(o, SUB)])
            return 0

        lax.fori_loop(0, RPT // SUB, combine_block, 0)

        if k < K:
            @pl.when(c != 0)
            def _():
                pltpu.sync_copy(zeros_hbm.at[pl.ds(base, RPT)],
                                acc.at[pl.ds(base, RPT)])
        plsc.subcore_barrier()


DW = 8


@functools.cache
def _make_sc_degree():
    sc_mesh = plsc.VectorSubcoreMesh(
        core_axis_name="c", subcore_axis_name="s", num_cores=NC, num_subcores=NS
    )
    return pl.kernel(
        _sc_degree_body,
        out_type=jax.ShapeDtypeStruct((NC * N_PAD, DW), jnp.float32),
        mesh=sc_mesh,
        scratch_types=[
            pltpu.VMEM((CPT, CH), jnp.int32),
            pltpu.VMEM((CH, DW), jnp.float32),
            pltpu.VMEM_SHARED((N_PAD, DW), jnp.float32),
        ],
        compiler_params=pltpu.CompilerParams(use_tc_tiling_on_sc=False),
    )


def _sc_degree(*args):
    return _make_sc_degree()(*args)


def _sc_degree_body(ones_hbm, zeros_hbm, cols_hbm, parts_hbm,
                    col_idx, ones_v, acc):
    c = lax.axis_index("c")
    s = lax.axis_index("s")
    w = s * NC + c

    pltpu.sync_copy(cols_hbm.at[pl.ds(w * CPT, CPT)], col_idx)
    pltpu.sync_copy(ones_hbm.at[pl.ds(0, CH)], ones_v)

    @pl.when(c == 0)
    def _():
        pltpu.sync_copy(ones_hbm.at[pl.ds(s * RPT, RPT)],
                        acc.at[pl.ds(s * RPT, RPT)])

    @pl.when(c != 0)
    def _():
        pltpu.sync_copy(zeros_hbm.at[pl.ds(s * RPT, RPT)],
                        acc.at[pl.ds(s * RPT, RPT)])

    plsc.subcore_barrier()

    def chunk(i, _):
        pltpu.sync_copy(ones_v, acc.at[col_idx.at[i]], add=True)
        return 0

    lax.fori_loop(0, CPT, chunk, 0)

    plsc.subcore_barrier()

    pltpu.sync_copy(acc.at[pl.ds(s * RPT, RPT)],
                    parts_hbm.at[pl.ds(c * N_PAD + s * RPT, RPT)])


def _mlp_body(x_ref, w1_ref, b1_ref, w2_ref, b2_ref, h_ref):
    a = jnp.dot(x_ref[...], w1_ref[...], preferred_element_type=jnp.float32)
    a = jnp.maximum(a + b1_ref[...], 0.0)
    h_ref[...] = (
        jnp.dot(a, w2_ref[...], preferred_element_type=jnp.float32) + b2_ref[...]
    )


def _dinv_body(p0_ref, p1_ref, h_ref, dinv_ref, s0_ref):
    deg = p0_ref[...] + p1_ref[...]
    dinv = jnp.where(deg > 0.0, lax.rsqrt(deg), 0.0)[:, 0:1]
    dinv_ref[...] = dinv * jnp.ones((1, D_OUT), jnp.float32)
    s0_ref[...] = dinv * h_ref[...]


def _combine_body(p0_ref, p1_ref, dinv_ref, cur_ref, s_ref):
    t = dinv_ref[...] * (p0_ref[...] + p1_ref[...])
    cur_ref[...] = t
    s_ref[...] = dinv_ref[...] * t


def _retention_body(*refs):
    pred_refs = refs[: K + 1]
    wp_ref, bp_ref, out_ref = refs[K + 1], refs[K + 2], refs[K + 3]
    acc = jnp.zeros(out_ref.shape, out_ref.dtype)
    for p_ref in pred_refs:
        p = p_ref[...]
        sc = jnp.sum(p * wp_ref[...], axis=1, keepdims=True) + bp_ref[...]
        sg = 1.0 / (1.0 + jnp.exp(-sc))
        acc = acc + sg * p
    out_ref[...] = acc


_BN = 80
_NBP = N_PAD // _BN
_NBN = N // _BN


def _row_spec(d):
    return pl.BlockSpec((_BN, d), lambda i: (i, 0))


def _p1_spec():
    return pl.BlockSpec((_BN, D_OUT), lambda i: (i + _NBP, 0))


def _full_spec(r, c):
    return pl.BlockSpec((r, c), lambda i: (0, 0))


def kernel(x, edge_index, W1, b1, W2, b2, Wp, bp):
    f32 = jnp.float32
    row = edge_index[0]
    col = edge_index[1]
    colp = jnp.where(row == col, DUMMY, col).astype(jnp.int32)
    pad = E_PAD - E
    rows_full = jnp.concatenate([row.astype(jnp.int32),
                                 jnp.zeros((pad,), jnp.int32)]
                                ).reshape(NW * CPT, CH)
    cols_full = jnp.concatenate([colp, jnp.full((pad,), DUMMY, jnp.int32)]
                                ).reshape(NW * CPT, CH)
    zeros_pd = jnp.zeros((N_PAD, D_OUT), f32)
    zeros_dw = jnp.zeros((N_PAD, DW), f32)
    ones_dw = jnp.ones((N_PAD, DW), f32)

    h = pl.pallas_call(
        _mlp_body,
        grid=(_NBN,),
        in_specs=[
            _row_spec(D_IN),
            _full_spec(D_IN, D_HID),
            _full_spec(1, D_HID),
            _full_spec(D_HID, D_OUT),
            _full_spec(1, D_OUT),
        ],
        out_specs=_row_spec(D_OUT),
        out_shape=jax.ShapeDtypeStruct((N, D_OUT), f32),
    )(x, W1, b1.reshape(1, D_HID), W2, b2.reshape(1, D_OUT))
    h_pd = jnp.pad(h, ((0, N_PAD - N), (0, 0)))

    deg_parts = _sc_degree(ones_dw, zeros_dw, cols_full)

    dinv, cur_s = pl.pallas_call(
        _dinv_body,
        grid=(_NBP,),
        in_specs=[
            pl.BlockSpec((_BN, DW), lambda i: (i, 0)),
            pl.BlockSpec((_BN, DW), lambda i: (i + _NBP, 0)),
            _row_spec(D_OUT),
        ],
        out_specs=[_row_spec(D_OUT), _row_spec(D_OUT)],
        out_shape=[
            jax.ShapeDtypeStruct((N_PAD, D_OUT), f32),
            jax.ShapeDtypeStruct((N_PAD, D_OUT), f32),
        ],
    )(deg_parts, deg_parts, h_pd)

    preds_out, _parts, _flags = _sc_fused(cur_s, zeros_pd, dinv,
                                          rows_full, cols_full)
    preds = [h]

    sec = N_PAD // _BN

    def _pred_spec(k):
        return pl.BlockSpec((_BN, D_OUT), lambda i, kk=k: (i + kk * sec, 0))

    out = pl.pallas_call(
        _retention_body,
        grid=(_NBN,),
        in_specs=[_row_spec(D_OUT)]
        + [_pred_spec(k) for k in range(K)]
        + [_full_spec(1, D_OUT), _full_spec(1, 1)],
        out_specs=_row_spec(D_OUT),
        out_shape=jax.ShapeDtypeStruct((N, D_OUT), f32),
    )(h, *([preds_out] * K), Wp.reshape(1, D_OUT), bp.reshape(1, 1))
    return out
